# baseline (speedup 1.0000x reference)
import functools

import jax
import jax.numpy as jnp
from jax import lax
from jax.experimental import pallas as pl
from jax.experimental.pallas import tpu as pltpu
from jax.experimental.pallas import tpu_sc as plsc

T, D, FF, E, K = 2048, 1024, 2048, 8, 2
BLK = 256
N_PAD = T * K + E * BLK
NB = N_PAD // BLK

NC, NS = 2, 16
NW = NC * NS
TPW = T // NW
CHUNK = 32


def _gate_body(x_ref, wg_ref, bg_ref, idx_ref, vals_ref, dest_ref, be_ref,
               v0x_ref, v1x_ref):
    xf = x_ref[...]
    logits = lax.dot_general(xf, wg_ref[...], (((1,), (1,)), ((), ())),
                             preferred_element_type=jnp.float32)
    logits = logits + bg_ref[...]
    m = jnp.max(logits, axis=1, keepdims=True)
    p = jnp.exp(logits - m)
    scores = p / jnp.sum(p, axis=1, keepdims=True)

    iota_e = lax.broadcasted_iota(jnp.int32, (T, E), 1)
    m1 = jnp.max(scores, axis=1, keepdims=True)
    i1 = jnp.min(jnp.where(scores == m1, iota_e, E), axis=1, keepdims=True)
    sel1 = iota_e == i1
    masked = jnp.where(sel1, -1.0, scores)
    m2 = jnp.max(masked, axis=1, keepdims=True)
    i2 = jnp.min(jnp.where(masked == m2, iota_e, E), axis=1, keepdims=True)
    sel2 = iota_e == i2

    idx_ref[...] = jnp.concatenate([i1, i2], axis=1)
    vals_ref[...] = jnp.concatenate([m1, m2], axis=1)
    zeros16 = jnp.zeros((T, 128), jnp.float32)
    v0x_ref[...] = m1 + zeros16
    v1x_ref[...] = m2 + zeros16

    m0f = sel1.astype(jnp.float32)
    m1f = sel2.astype(jnp.float32)
    rowsum = m0f + m1f
    ti = lax.broadcasted_iota(jnp.int32, (T, T), 0)
    tj = lax.broadcasted_iota(jnp.int32, (T, T), 1)
    tri = (tj < ti).astype(jnp.float32)
    cum_excl = lax.dot_general(tri, rowsum, (((1,), (0,)), ((), ())),
                               preferred_element_type=jnp.float32)
    counts = jnp.sum(rowsum, axis=0, keepdims=True)
    cnt_pad = jnp.floor((counts + (BLK - 1)) * (1.0 / BLK)) * BLK
    ei = lax.broadcasted_iota(jnp.int32, (E, E), 0)
    ej = lax.broadcasted_iota(jnp.int32, (E, E), 1)
    tri_e = (ei < ej).astype(jnp.float32)
    pad_off = lax.dot_general(cnt_pad, tri_e, (((1,), (0,)), ((), ())),
                              preferred_element_type=jnp.float32)
    base = pad_off + cum_excl
    d0 = jnp.sum(jnp.where(sel1, base, 0.0), axis=1, keepdims=True)
    d1 = jnp.sum(jnp.where(sel2, base, 0.0), axis=1, keepdims=True)
    dest_ref[...] = jnp.concatenate([d0, d1], axis=1).astype(jnp.int32)

    pos = lax.broadcasted_iota(jnp.int32, (NB, 1), 0).astype(jnp.float32) * BLK
    owners = jnp.sum((pad_off <= pos).astype(jnp.int32), axis=1,
                     keepdims=True) - 1
    prev = jnp.concatenate(
        [jnp.full((1, 1), -1, jnp.int32), owners[:-1]], axis=0)
    first = (owners != prev).astype(jnp.float32)
    bi = lax.broadcasted_iota(jnp.int32, (NB, NB), 0)
    bj = lax.broadcasted_iota(jnp.int32, (NB, NB), 1)
    tri_nb = (bj <= bi).astype(jnp.float32)
    run_rank = lax.dot_general(tri_nb, first, (((1,), (0,)), ((), ())),
                               preferred_element_type=jnp.float32) - 1.0
    slot3 = run_rank - 3.0 * jnp.floor(run_rank * (1.0 / 3.0))
    present = (counts > 0).astype(jnp.float32)
    rank_e = lax.dot_general(present, tri_e, (((1,), (0,)), ((), ())),
                             preferred_element_type=jnp.float32)
    n_runs = jnp.sum(present, axis=1, keepdims=True)
    r2 = run_rank + 2.0
    ef = lax.broadcasted_iota(jnp.int32, (1, E), 1).astype(jnp.float32)
    match = (rank_e == r2) & (present > 0)
    nexte = jnp.sum(jnp.where(match, ef, 0.0), axis=1, keepdims=True)
    nexte = jnp.where(r2 >= n_runs, -1.0, nexte)
    ne1 = jnp.sum(jnp.where((rank_e == 1.0) & (present > 0), ef, 0.0),
                  axis=1, keepdims=True)
    nb_real = jnp.sum(cnt_pad, axis=1, keepdims=True) * (1.0 / BLK)
    pad8 = jnp.zeros((8, 1), jnp.float32)
    tail = jnp.concatenate([nb_real, ne1, jnp.zeros((6, 1), jnp.float32)],
                           axis=0)
    ownf = owners.astype(jnp.float32)
    sp = jnp.concatenate(
        [ownf, pad8, first, pad8, nexte, pad8, slot3, tail], axis=0)
    be_ref[...] = sp.astype(jnp.int32)


def _gate(xf, Wg, bg):
    return pl.pallas_call(
        _gate_body,
        out_shape=(
            jax.ShapeDtypeStruct((T, K), jnp.int32),
            jax.ShapeDtypeStruct((T, K), jnp.float32),
            jax.ShapeDtypeStruct((T, K), jnp.int32),
            jax.ShapeDtypeStruct((128, 1), jnp.int32),
            jax.ShapeDtypeStruct((T, 128), jnp.float32),
            jax.ShapeDtypeStruct((T, 128), jnp.float32),
        ),
    )(xf, Wg, bg.reshape(1, E))


def _dispatch_body(x_hbm, d0_hbm, d1_hbm, v0x_hbm, v1x_hbm, xs_hbm, wx_hbm,
                   rows_v, i0_v, i1_v, w0_v, w1_v, sem):
    wid = lax.axis_index("s") * NC + lax.axis_index("c")
    t0 = wid * TPW
    sl = pl.ds(t0, TPW)
    loads = [
        pltpu.async_copy(x_hbm.at[sl], rows_v, sem),
        pltpu.async_copy(d0_hbm.at[sl], i0_v, sem),
        pltpu.async_copy(d1_hbm.at[sl], i1_v, sem),
        pltpu.async_copy(v0x_hbm.at[sl], w0_v, sem),
        pltpu.async_copy(v1x_hbm.at[sl], w1_v, sem),
    ]
    for ld in loads:
        ld.wait()
    c0 = pltpu.async_copy(rows_v, xs_hbm.at[i0_v], sem)
    c1 = pltpu.async_copy(rows_v, xs_hbm.at[i1_v], sem)
    c2 = pltpu.async_copy(w0_v, wx_hbm.at[i0_v], sem)
    c3 = pltpu.async_copy(w1_v, wx_hbm.at[i1_v], sem)
    c0.wait()
    c1.wait()
    c2.wait()
    c3.wait()


@functools.cache
def _make_dispatch():
    return pl.kernel(
        _dispatch_body,
        out_type=(
            jax.ShapeDtypeStruct((N_PAD, D), jnp.float32),
            jax.ShapeDtypeStruct((N_PAD, 128), jnp.float32),
        ),
        mesh=plsc.VectorSubcoreMesh(core_axis_name="c", subcore_axis_name="s",
                                    num_cores=NC, num_subcores=NS),
        scratch_types=[
            pltpu.VMEM((TPW, D), jnp.float32),
            pltpu.VMEM((TPW,), jnp.int32),
            pltpu.VMEM((TPW,), jnp.int32),
            pltpu.VMEM((TPW, 128), jnp.float32),
            pltpu.VMEM((TPW, 128), jnp.float32),
            pltpu.SemaphoreType.DMA,
        ],
    )


def _ffn_body(sp_ref, xs_ref, w1_hbm, b1_ref, w2_hbm, b2_ref, wx_ref, ys_ref,
              w1b, w2b, sem1, sem2):
    b = pl.program_id(0)
    be = sp_ref[b]
    fi = sp_ref[32 + b]
    nx = sp_ref[64 + b]
    sl = sp_ref[96 + b]
    nbr = sp_ref[120]

    def issue(e, s):
        pltpu.make_async_copy(w1_hbm.at[e], w1b.at[s], sem1.at[s]).start()
        pltpu.make_async_copy(w2_hbm.at[e], w2b.at[s], sem2.at[s]).start()

    def compute(w1v, w2v):
        h = lax.dot_general(xs_ref[...], w1v, (((1,), (1,)), ((), ())),
                            preferred_element_type=jnp.float32)
        h = h + b1_ref[0]
        h = jnp.where(h >= 0, h, 0.1 * h)
        y = lax.dot_general(h, w2v, (((1,), (1,)), ((), ())),
                            preferred_element_type=jnp.float32)
        ys_ref[...] = (y + b2_ref[0]) * wx_ref[:, 0:1]

    @pl.when(b < nbr)
    def _():
        @pl.when(b == 0)
        def _():
            issue(be, 0)
            issue(sp_ref[121], 1)

        @pl.when(fi == 1)
        def _():
            pltpu.make_async_copy(w1_hbm.at[be], w1b.at[sl], sem1.at[sl]).wait()
            pltpu.make_async_copy(w2_hbm.at[be], w2b.at[sl], sem2.at[sl]).wait()

            @pl.when(nx >= 0)
            def _():
                issue(nx, lax.rem(sl + 2, 3))

        @pl.when(sl == 0)
        def _():
            compute(w1b[0], w2b[0])

        @pl.when(sl == 1)
        def _():
            compute(w1b[1], w2b[1])

        @pl.when(sl == 2)
        def _():
            compute(w1b[2], w2b[2])


def _ffn(sp, xs, W1, b1, W2, b2, wx):
    grid_spec = pltpu.PrefetchScalarGridSpec(
        num_scalar_prefetch=1,
        grid=(NB,),
        in_specs=[
            pl.BlockSpec((BLK, D), lambda b, sp: (b, 0)),
            pl.BlockSpec(memory_space=pl.ANY),
            pl.BlockSpec((1, 1, FF), lambda b, sp: (sp[b], 0, 0)),
            pl.BlockSpec(memory_space=pl.ANY),
            pl.BlockSpec((1, 1, D), lambda b, sp: (sp[b], 0, 0)),
            pl.BlockSpec((BLK, 128), lambda b, sp: (b, 0)),
        ],
        out_specs=pl.BlockSpec((BLK, D), lambda b, sp: (b, 0)),
        scratch_shapes=[
            pltpu.VMEM((3, FF, D), jnp.float32),
            pltpu.VMEM((3, D, FF), jnp.float32),
            pltpu.SemaphoreType.DMA((3,)),
            pltpu.SemaphoreType.DMA((3,)),
        ],
    )
    return pl.pallas_call(
        _ffn_body,
        grid_spec=grid_spec,
        out_shape=jax.ShapeDtypeStruct((N_PAD, D), jnp.float32),
    )(sp, xs, W1, b1.reshape(E, 1, FF), W2, b2.reshape(E, 1, D), wx)


SUB = 16
NSUB = TPW // SUB


def _combine_body(ys_hbm, d0_hbm, d1_hbm, out_hbm,
                  g0a, g1a, g0b, g1b, i0_v, i1_v, sem0, sem1):
    wid = lax.axis_index("s") * NC + lax.axis_index("c")
    t0 = wid * TPW
    pltpu.sync_copy(d0_hbm.at[pl.ds(t0, TPW)], i0_v)
    pltpu.sync_copy(d1_hbm.at[pl.ds(t0, TPW)], i1_v)
    bufs = [(g0a, g1a, sem0), (g0b, g1b, sem1)]

    def issue(c):
        g0, g1, sem = bufs[c % 2]
        sl = pl.ds(c * SUB, SUB)
        pltpu.async_copy(ys_hbm.at[i0_v.at[sl]], g0, sem)
        pltpu.async_copy(ys_hbm.at[i1_v.at[sl]], g1, sem)

    def drain(c):
        g0, g1, sem = bufs[c % 2]
        sl = pl.ds(c * SUB, SUB)
        pltpu.make_async_copy(ys_hbm.at[i0_v.at[sl]], g0, sem).wait()
        pltpu.make_async_copy(ys_hbm.at[i1_v.at[sl]], g1, sem).wait()

    issue(0)
    for c in range(NSUB):
        g0, g1, _ = bufs[c % 2]
        drain(c)
        if c + 1 < NSUB:
            issue(c + 1)

        def row_body(r, carry):
            for cc in range(D // 16):
                sl = pl.ds(cc * 16, 16)
                g0[r, sl] = g0[r, sl] + g1[r, sl]
            return carry

        lax.fori_loop(0, SUB, row_body, 0)
        pltpu.sync_copy(g0, out_hbm.at[pl.ds(t0 + c * SUB, SUB)])


@functools.cache
def _make_combine():
    return pl.kernel(
        _combine_body,
        out_type=jax.ShapeDtypeStruct((T, D), jnp.float32),
        mesh=plsc.VectorSubcoreMesh(core_axis_name="c", subcore_axis_name="s",
                                    num_cores=NC, num_subcores=NS),
        scratch_types=[
            pltpu.VMEM((SUB, D), jnp.float32),
            pltpu.VMEM((SUB, D), jnp.float32),
            pltpu.VMEM((SUB, D), jnp.float32),
            pltpu.VMEM((SUB, D), jnp.float32),
            pltpu.VMEM((TPW,), jnp.int32),
            pltpu.VMEM((TPW,), jnp.int32),
            pltpu.SemaphoreType.DMA,
            pltpu.SemaphoreType.DMA,
        ],
    )


def kernel(x, Wg, bg, W1, b1, W2, b2):
    b, s, d = x.shape
    xf = x.reshape(T, D)
    topk_idx, topk_vals, dest, be, v0x, v1x = _gate(xf, Wg, bg)
    d0, d1 = dest[:, 0], dest[:, 1]
    xs, wx = _make_dispatch()(xf, d0, d1, v0x, v1x)
    ys = _ffn(be[:, 0], xs, W1, b1, W2, b2, wx)
    out = _make_combine()(ys, d0, d1)
    return out.reshape(b, s, d), topk_idx, topk_vals


# speedup vs baseline: 1.0131x; 1.0131x over previous
import functools

import jax
import jax.numpy as jnp
from jax import lax
from jax.experimental import pallas as pl
from jax.experimental.pallas import tpu as pltpu
from jax.experimental.pallas import tpu_sc as plsc

T, D, FF, E, K = 2048, 1024, 2048, 8, 2
BLK = 256
N_PAD = T * K + E * BLK
NB = N_PAD // BLK

NC, NS = 2, 16
NW = NC * NS
TPW = T // NW
CHUNK = 32


def _gate_body(x_ref, wg_ref, bg_ref, idx_ref, vals_ref, dest_ref, be_ref,
               v0x_ref, v1x_ref):
    xf = x_ref[...]
    logits = lax.dot_general(xf, wg_ref[...], (((1,), (1,)), ((), ())),
                             preferred_element_type=jnp.float32)
    logits = logits + bg_ref[...]
    m = jnp.max(logits, axis=1, keepdims=True)
    p = jnp.exp(logits - m)
    scores = p / jnp.sum(p, axis=1, keepdims=True)

    iota_e = lax.broadcasted_iota(jnp.int32, (T, E), 1)
    m1 = jnp.max(scores, axis=1, keepdims=True)
    i1 = jnp.min(jnp.where(scores == m1, iota_e, E), axis=1, keepdims=True)
    sel1 = iota_e == i1
    masked = jnp.where(sel1, -1.0, scores)
    m2 = jnp.max(masked, axis=1, keepdims=True)
    i2 = jnp.min(jnp.where(masked == m2, iota_e, E), axis=1, keepdims=True)
    sel2 = iota_e == i2

    idx_ref[...] = jnp.concatenate([i1, i2], axis=1)
    vals_ref[...] = jnp.concatenate([m1, m2], axis=1)
    zeros16 = jnp.zeros((T, 128), jnp.float32)
    v0x_ref[...] = m1 + zeros16
    v1x_ref[...] = m2 + zeros16

    m0f = sel1.astype(jnp.float32)
    m1f = sel2.astype(jnp.float32)
    rowsum = m0f + m1f
    ti = lax.broadcasted_iota(jnp.int32, (T, T), 0)
    tj = lax.broadcasted_iota(jnp.int32, (T, T), 1)
    tri = (tj < ti).astype(jnp.float32)
    cum_excl = lax.dot_general(tri, rowsum, (((1,), (0,)), ((), ())),
                               preferred_element_type=jnp.float32)
    counts = jnp.sum(rowsum, axis=0, keepdims=True)
    cnt_pad = jnp.floor((counts + (BLK - 1)) * (1.0 / BLK)) * BLK
    ei = lax.broadcasted_iota(jnp.int32, (E, E), 0)
    ej = lax.broadcasted_iota(jnp.int32, (E, E), 1)
    tri_e = (ei < ej).astype(jnp.float32)
    pad_off = lax.dot_general(cnt_pad, tri_e, (((1,), (0,)), ((), ())),
                              preferred_element_type=jnp.float32)
    base = pad_off + cum_excl
    d0 = jnp.sum(jnp.where(sel1, base, 0.0), axis=1, keepdims=True)
    d1 = jnp.sum(jnp.where(sel2, base, 0.0), axis=1, keepdims=True)
    dest_ref[...] = jnp.concatenate([d0, d1], axis=1).astype(jnp.int32)

    pos = lax.broadcasted_iota(jnp.int32, (NB, 1), 0).astype(jnp.float32) * BLK
    owners = jnp.sum((pad_off <= pos).astype(jnp.int32), axis=1,
                     keepdims=True) - 1
    prev = jnp.concatenate(
        [jnp.full((1, 1), -1, jnp.int32), owners[:-1]], axis=0)
    first = (owners != prev).astype(jnp.float32)
    bi = lax.broadcasted_iota(jnp.int32, (NB, NB), 0)
    bj = lax.broadcasted_iota(jnp.int32, (NB, NB), 1)
    tri_nb = (bj <= bi).astype(jnp.float32)
    run_rank = lax.dot_general(tri_nb, first, (((1,), (0,)), ((), ())),
                               preferred_element_type=jnp.float32) - 1.0
    parity = run_rank - 2.0 * jnp.floor(run_rank * 0.5)
    present = (counts > 0).astype(jnp.float32)
    rank_e = lax.dot_general(present, tri_e, (((1,), (0,)), ((), ())),
                             preferred_element_type=jnp.float32)
    n_runs = jnp.sum(present, axis=1, keepdims=True)
    r1 = run_rank + 1.0
    ef = lax.broadcasted_iota(jnp.int32, (1, E), 1).astype(jnp.float32)
    match = (rank_e == r1) & (present > 0)
    nexte = jnp.sum(jnp.where(match, ef, 0.0), axis=1, keepdims=True)
    nexte = jnp.where(r1 >= n_runs, -1.0, nexte)
    nb_real = jnp.sum(cnt_pad, axis=1, keepdims=True) * (1.0 / BLK)
    pad8 = jnp.zeros((8, 1), jnp.float32)
    nbr_col = jnp.broadcast_to(nb_real, (8, 1))
    ownf = owners.astype(jnp.float32)
    sp = jnp.concatenate(
        [ownf, pad8, first, pad8, nexte, pad8, parity, nbr_col], axis=0)
    be_ref[...] = sp.astype(jnp.int32)


def _gate(xf, Wg, bg):
    return pl.pallas_call(
        _gate_body,
        out_shape=(
            jax.ShapeDtypeStruct((T, K), jnp.int32),
            jax.ShapeDtypeStruct((T, K), jnp.float32),
            jax.ShapeDtypeStruct((T, K), jnp.int32),
            jax.ShapeDtypeStruct((128, 1), jnp.int32),
            jax.ShapeDtypeStruct((T, 128), jnp.float32),
            jax.ShapeDtypeStruct((T, 128), jnp.float32),
        ),
    )(xf, Wg, bg.reshape(1, E))


def _dispatch_body(x_hbm, d0_hbm, d1_hbm, v0x_hbm, v1x_hbm, xs_hbm, wx_hbm,
                   rows_v, i0_v, i1_v, w0_v, w1_v, sem):
    wid = lax.axis_index("s") * NC + lax.axis_index("c")
    t0 = wid * TPW
    sl = pl.ds(t0, TPW)
    loads = [
        pltpu.async_copy(x_hbm.at[sl], rows_v, sem),
        pltpu.async_copy(d0_hbm.at[sl], i0_v, sem),
        pltpu.async_copy(d1_hbm.at[sl], i1_v, sem),
        pltpu.async_copy(v0x_hbm.at[sl], w0_v, sem),
        pltpu.async_copy(v1x_hbm.at[sl], w1_v, sem),
    ]
    for ld in loads:
        ld.wait()
    c0 = pltpu.async_copy(rows_v, xs_hbm.at[i0_v], sem)
    c1 = pltpu.async_copy(rows_v, xs_hbm.at[i1_v], sem)
    c2 = pltpu.async_copy(w0_v, wx_hbm.at[i0_v], sem)
    c3 = pltpu.async_copy(w1_v, wx_hbm.at[i1_v], sem)
    c0.wait()
    c1.wait()
    c2.wait()
    c3.wait()


@functools.cache
def _make_dispatch():
    return pl.kernel(
        _dispatch_body,
        out_type=(
            jax.ShapeDtypeStruct((N_PAD, D), jnp.float32),
            jax.ShapeDtypeStruct((N_PAD, 128), jnp.float32),
        ),
        mesh=plsc.VectorSubcoreMesh(core_axis_name="c", subcore_axis_name="s",
                                    num_cores=NC, num_subcores=NS),
        scratch_types=[
            pltpu.VMEM((TPW, D), jnp.float32),
            pltpu.VMEM((TPW,), jnp.int32),
            pltpu.VMEM((TPW,), jnp.int32),
            pltpu.VMEM((TPW, 128), jnp.float32),
            pltpu.VMEM((TPW, 128), jnp.float32),
            pltpu.SemaphoreType.DMA,
        ],
    )


def _ffn_body(sp_ref, xs_ref, w1_hbm, b1_ref, w2_hbm, b2_ref, wx_ref, ys_ref,
              w1b, w2b, sem1, sem2):
    b = pl.program_id(0)
    be = sp_ref[b]
    fi = sp_ref[32 + b]
    nx = sp_ref[64 + b]
    sl = sp_ref[96 + b]
    nbr = sp_ref[120]

    def issue(e, s):
        pltpu.make_async_copy(w1_hbm.at[e], w1b.at[s], sem1.at[s]).start()
        pltpu.make_async_copy(w2_hbm.at[e], w2b.at[s], sem2.at[s]).start()

    def compute(w1v, w2v):
        h = lax.dot_general(xs_ref[...], w1v, (((1,), (1,)), ((), ())),
                            preferred_element_type=jnp.float32)
        h = h + b1_ref[0]
        h = jnp.where(h >= 0, h, 0.1 * h)
        y = lax.dot_general(h, w2v, (((1,), (1,)), ((), ())),
                            preferred_element_type=jnp.float32)
        ys_ref[...] = (y + b2_ref[0]) * wx_ref[:, 0:1]

    @pl.when(b < nbr)
    def _():
        @pl.when(b == 0)
        def _():
            issue(be, 0)

        @pl.when(fi == 1)
        def _():
            pltpu.make_async_copy(w1_hbm.at[be], w1b.at[sl], sem1.at[sl]).wait()
            pltpu.make_async_copy(w2_hbm.at[be], w2b.at[sl], sem2.at[sl]).wait()

            @pl.when(nx >= 0)
            def _():
                issue(nx, 1 - sl)

        @pl.when(sl == 0)
        def _():
            compute(w1b[0], w2b[0])

        @pl.when(sl == 1)
        def _():
            compute(w1b[1], w2b[1])


def _ffn(sp, xs, W1, b1, W2, b2, wx):
    grid_spec = pltpu.PrefetchScalarGridSpec(
        num_scalar_prefetch=1,
        grid=(NB,),
        in_specs=[
            pl.BlockSpec((BLK, D), lambda b, sp: (b, 0)),
            pl.BlockSpec(memory_space=pl.ANY),
            pl.BlockSpec((1, 1, FF), lambda b, sp: (sp[b], 0, 0)),
            pl.BlockSpec(memory_space=pl.ANY),
            pl.BlockSpec((1, 1, D), lambda b, sp: (sp[b], 0, 0)),
            pl.BlockSpec((BLK, 128), lambda b, sp: (b, 0)),
        ],
        out_specs=pl.BlockSpec((BLK, D), lambda b, sp: (b, 0)),
        scratch_shapes=[
            pltpu.VMEM((2, FF, D), jnp.float32),
            pltpu.VMEM((2, D, FF), jnp.float32),
            pltpu.SemaphoreType.DMA((2,)),
            pltpu.SemaphoreType.DMA((2,)),
        ],
    )
    return pl.pallas_call(
        _ffn_body,
        grid_spec=grid_spec,
        out_shape=jax.ShapeDtypeStruct((N_PAD, D), jnp.float32),
    )(sp, xs, W1, b1.reshape(E, 1, FF), W2, b2.reshape(E, 1, D), wx)


SUB = 16
NSUB = TPW // SUB


def _combine_body(ys_hbm, d0_hbm, d1_hbm, out_hbm,
                  g0a, g1a, g0b, g1b, i0_v, i1_v, sem0, sem1):
    wid = lax.axis_index("s") * NC + lax.axis_index("c")
    t0 = wid * TPW
    pltpu.sync_copy(d0_hbm.at[pl.ds(t0, TPW)], i0_v)
    pltpu.sync_copy(d1_hbm.at[pl.ds(t0, TPW)], i1_v)
    bufs = [(g0a, g1a, sem0), (g0b, g1b, sem1)]

    def issue(c):
        g0, g1, sem = bufs[c % 2]
        sl = pl.ds(c * SUB, SUB)
        pltpu.async_copy(ys_hbm.at[i0_v.at[sl]], g0, sem)
        pltpu.async_copy(ys_hbm.at[i1_v.at[sl]], g1, sem)

    def drain(c):
        g0, g1, sem = bufs[c % 2]
        sl = pl.ds(c * SUB, SUB)
        pltpu.make_async_copy(ys_hbm.at[i0_v.at[sl]], g0, sem).wait()
        pltpu.make_async_copy(ys_hbm.at[i1_v.at[sl]], g1, sem).wait()

    issue(0)
    for c in range(NSUB):
        g0, g1, _ = bufs[c % 2]
        drain(c)
        if c + 1 < NSUB:
            issue(c + 1)

        def row_body(r, carry):
            for cc in range(D // 16):
                sl = pl.ds(cc * 16, 16)
                g0[r, sl] = g0[r, sl] + g1[r, sl]
            return carry

        lax.fori_loop(0, SUB, row_body, 0)
        pltpu.sync_copy(g0, out_hbm.at[pl.ds(t0 + c * SUB, SUB)])


@functools.cache
def _make_combine():
    return pl.kernel(
        _combine_body,
        out_type=jax.ShapeDtypeStruct((T, D), jnp.float32),
        mesh=plsc.VectorSubcoreMesh(core_axis_name="c", subcore_axis_name="s",
                                    num_cores=NC, num_subcores=NS),
        scratch_types=[
            pltpu.VMEM((SUB, D), jnp.float32),
            pltpu.VMEM((SUB, D), jnp.float32),
            pltpu.VMEM((SUB, D), jnp.float32),
            pltpu.VMEM((SUB, D), jnp.float32),
            pltpu.VMEM((TPW,), jnp.int32),
            pltpu.VMEM((TPW,), jnp.int32),
            pltpu.SemaphoreType.DMA,
            pltpu.SemaphoreType.DMA,
        ],
    )


def kernel(x, Wg, bg, W1, b1, W2, b2):
    b, s, d = x.shape
    xf = x.reshape(T, D)
    topk_idx, topk_vals, dest, be, v0x, v1x = _gate(xf, Wg, bg)
    d0, d1 = dest[:, 0], dest[:, 1]
    xs, wx = _make_dispatch()(xf, d0, d1, v0x, v1x)
    ys = _ffn(be[:, 0], xs, W1, b1, W2, b2, wx)
    out = _make_combine()(ys, d0, d1)
    return out.reshape(b, s, d), topk_idx, topk_vals
